# Initial kernel scaffold; baseline (speedup 1.0000x reference)
import jax
import jax.numpy as jnp
from jax.experimental import pallas as pl


def kernel(x, h, sign):
    raise NotImplementedError("write your pallas kernel here")


# speedup vs baseline: 2.7387x; 2.7387x over previous
import functools

import jax
import jax.numpy as jnp
from jax.experimental import pallas as pl
from jax.experimental.pallas import tpu as pltpu

M = 2048
BATCH_BLOCK = 512


def _body(h_ref, sign_ref, x_ref, out_ref, s_scratch):
    d = h_ref.shape[0]

    @pl.when(pl.program_id(0) == 0)
    def _build():
        bins = jax.lax.broadcasted_iota(jnp.int32, (d, M), 1)
        onehot = jnp.where(h_ref[...] == bins, sign_ref[...], 0.0)
        s_scratch[...] = onehot.astype(jnp.bfloat16)

    out_ref[...] = jnp.dot(
        x_ref[...].astype(jnp.bfloat16),
        s_scratch[...],
        preferred_element_type=jnp.float32,
    )


def kernel(x, h, sign):
    batch, d = x.shape
    h2 = h.astype(jnp.int32).reshape(d, 1)
    sign2 = sign.reshape(d, 1)

    grid = (batch // BATCH_BLOCK,)
    return pl.pallas_call(
        _body,
        grid=grid,
        in_specs=[
            pl.BlockSpec((d, 1), lambda b: (0, 0)),
            pl.BlockSpec((d, 1), lambda b: (0, 0)),
            pl.BlockSpec((BATCH_BLOCK, d), lambda b: (b, 0)),
        ],
        out_specs=pl.BlockSpec((BATCH_BLOCK, M), lambda b: (b, 0)),
        out_shape=jax.ShapeDtypeStruct((batch, M), x.dtype),
        scratch_shapes=[pltpu.VMEM((d, M), jnp.bfloat16)],
        compiler_params=pltpu.CompilerParams(
            dimension_semantics=("arbitrary",),
        ),
    )(h2, sign2, x)
